# baseline (speedup 1.0000x reference)
import jax
import jax.numpy as jnp
from jax.experimental import pallas as pl
from jax.experimental.pallas import tpu as pltpu

N = 10000
T = 8
IN = 128
OUT = 128
B = 2000


def _agg_kernel(nt_ref, x_ref, w_ref, b_ref, o_ref):
    x = jnp.maximum(x_ref[...], 0.0).astype(jnp.bfloat16)
    nt = nt_ref[...]
    ys = [jnp.dot(x, w_ref[t], preferred_element_type=jnp.float32) for t in range(T)]
    b0 = (nt & 1) == 1
    b1 = (nt & 2) == 2
    b2 = (nt & 4) == 4
    ys = [jnp.where(b0, ys[2 * i + 1], ys[2 * i]) for i in range(4)]
    ys = [jnp.where(b1, ys[2 * i + 1], ys[2 * i]) for i in range(2)]
    y = jnp.where(b2, ys[1], ys[0])
    onehot = (nt == jax.lax.broadcasted_iota(jnp.int32, (1, T), 1)).astype(jnp.float32)
    bias = jnp.dot(onehot, b_ref[...], preferred_element_type=jnp.float32)
    o_ref[...] = jnp.maximum(y + bias, 0.0)


def kernel(agg_msg, node_type, W_att, b_att):
    x = agg_msg.reshape(N, IN)
    W_att = W_att.astype(jnp.bfloat16)
    nt = node_type.astype(jnp.int32).reshape(N, 1)
    out = pl.pallas_call(
        _agg_kernel,
        grid=(N // B,),
        in_specs=[
            pl.BlockSpec((B, 1), lambda i: (i, 0)),
            pl.BlockSpec((B, IN), lambda i: (i, 0)),
            pl.BlockSpec((T, IN, OUT), lambda i: (0, 0, 0)),
            pl.BlockSpec((T, OUT), lambda i: (0, 0)),
        ],
        out_specs=pl.BlockSpec((B, OUT), lambda i: (i, 0)),
        out_shape=jax.ShapeDtypeStruct((N, OUT), jnp.float32),
    )(nt, x, W_att, b_att)
    return out


# speedup vs baseline: 1.0122x; 1.0122x over previous
import jax
import jax.numpy as jnp
from jax.experimental import pallas as pl
from jax.experimental.pallas import tpu as pltpu

N = 10000
T = 8
IN = 128
OUT = 128
B = 5000


def _agg_kernel(nt_ref, x_ref, w_ref, b_ref, o_ref):
    x = jnp.maximum(x_ref[...], 0.0)
    nt = nt_ref[...]
    ys = [jnp.dot(x, w_ref[t], preferred_element_type=jnp.float32) for t in range(T)]
    b0 = (nt & 1) == 1
    b1 = (nt & 2) == 2
    b2 = (nt & 4) == 4
    ys = [jnp.where(b0, ys[2 * i + 1], ys[2 * i]) for i in range(4)]
    ys = [jnp.where(b1, ys[2 * i + 1], ys[2 * i]) for i in range(2)]
    y = jnp.where(b2, ys[1], ys[0])
    onehot = (nt == jax.lax.broadcasted_iota(jnp.int32, (1, T), 1)).astype(jnp.float32)
    bias = jnp.dot(onehot, b_ref[...], preferred_element_type=jnp.float32)
    o_ref[...] = jnp.maximum(y + bias, 0.0)


def kernel(agg_msg, node_type, W_att, b_att):
    x = agg_msg.reshape(N, IN)
    nt = node_type.astype(jnp.int32).reshape(N, 1)
    out = pl.pallas_call(
        _agg_kernel,
        grid=(N // B,),
        in_specs=[
            pl.BlockSpec((B, 1), lambda i: (i, 0)),
            pl.BlockSpec((B, IN), lambda i: (i, 0)),
            pl.BlockSpec((T, IN, OUT), lambda i: (0, 0, 0)),
            pl.BlockSpec((T, OUT), lambda i: (0, 0)),
        ],
        out_specs=pl.BlockSpec((B, OUT), lambda i: (i, 0)),
        out_shape=jax.ShapeDtypeStruct((N, OUT), jnp.float32),
    )(nt, x, W_att, b_att)
    return out


# speedup vs baseline: 1.0149x; 1.0027x over previous
import jax
import jax.numpy as jnp
from jax.experimental import pallas as pl
from jax.experimental.pallas import tpu as pltpu

N = 10000
T = 8
IN = 128
OUT = 128
B = 1000


def _agg_kernel(nt_ref, x_ref, w_ref, b_ref, o_ref):
    x = jnp.maximum(x_ref[...], 0.0)
    nt = nt_ref[...]
    ys = [jnp.dot(x, w_ref[t], preferred_element_type=jnp.float32) for t in range(T)]
    b0 = (nt & 1) == 1
    b1 = (nt & 2) == 2
    b2 = (nt & 4) == 4
    ys = [jnp.where(b0, ys[2 * i + 1], ys[2 * i]) for i in range(4)]
    ys = [jnp.where(b1, ys[2 * i + 1], ys[2 * i]) for i in range(2)]
    y = jnp.where(b2, ys[1], ys[0])
    onehot = (nt == jax.lax.broadcasted_iota(jnp.int32, (1, T), 1)).astype(jnp.float32)
    bias = jnp.dot(onehot, b_ref[...], preferred_element_type=jnp.float32)
    o_ref[...] = jnp.maximum(y + bias, 0.0)


def kernel(agg_msg, node_type, W_att, b_att):
    x = agg_msg.reshape(N, IN)
    nt = node_type.astype(jnp.int32).reshape(N, 1)
    out = pl.pallas_call(
        _agg_kernel,
        grid=(N // B,),
        in_specs=[
            pl.BlockSpec((B, 1), lambda i: (i, 0)),
            pl.BlockSpec((B, IN), lambda i: (i, 0)),
            pl.BlockSpec((T, IN, OUT), lambda i: (0, 0, 0)),
            pl.BlockSpec((T, OUT), lambda i: (0, 0)),
        ],
        out_specs=pl.BlockSpec((B, OUT), lambda i: (i, 0)),
        out_shape=jax.ShapeDtypeStruct((N, OUT), jnp.float32),
    )(nt, x, W_att, b_att)
    return out


# speedup vs baseline: 1.2865x; 1.2676x over previous
import jax
import jax.numpy as jnp
from jax.experimental import pallas as pl
from jax.experimental.pallas import tpu as pltpu

N = 10000
T = 8
IN = 128
OUT = 128
B = 2000


def _agg_kernel(nt_ref, x_ref, w_ref, b_ref, o_ref):
    x = jnp.maximum(x_ref[...], 0.0)
    nt = nt_ref[...]
    ys = [jnp.dot(x, w_ref[t], preferred_element_type=jnp.float32) for t in range(T)]
    b0 = (nt & 1) == 1
    b1 = (nt & 2) == 2
    b2 = (nt & 4) == 4
    ys = [jnp.where(b0, ys[2 * i + 1], ys[2 * i]) for i in range(4)]
    ys = [jnp.where(b1, ys[2 * i + 1], ys[2 * i]) for i in range(2)]
    y = jnp.where(b2, ys[1], ys[0])
    onehot = (nt == jax.lax.broadcasted_iota(jnp.int32, (1, T), 1)).astype(jnp.float32)
    bias = jnp.dot(onehot, b_ref[...], preferred_element_type=jnp.float32)
    o_ref[...] = jnp.maximum(y + bias, 0.0)


def kernel(agg_msg, node_type, W_att, b_att):
    x = agg_msg.reshape(N, IN)
    nt = node_type.astype(jnp.int32).reshape(N, 1)
    out = pl.pallas_call(
        _agg_kernel,
        grid=(N // B,),
        in_specs=[
            pl.BlockSpec((B, 1), lambda i: (i, 0)),
            pl.BlockSpec((B, IN), lambda i: (i, 0)),
            pl.BlockSpec((T, IN, OUT), lambda i: (0, 0, 0)),
            pl.BlockSpec((T, OUT), lambda i: (0, 0)),
        ],
        out_specs=pl.BlockSpec((B, OUT), lambda i: (i, 0)),
        out_shape=jax.ShapeDtypeStruct((N, OUT), jnp.float32),
    )(nt, x, W_att, b_att)
    return out


# speedup vs baseline: 1.2880x; 1.0012x over previous
import jax
import jax.numpy as jnp
from jax.experimental import pallas as pl

N = 10000
T = 8
IN = 128
OUT = 128
B = 2000


def _agg_kernel(nt_ref, x_ref, w_ref, b_ref, o_ref):
    x = jnp.maximum(x_ref[...], 0.0)
    nt = nt_ref[...]
    ys = [jnp.dot(x, w_ref[t], preferred_element_type=jnp.float32) for t in range(T)]
    b0 = (nt & 1) == 1
    b1 = (nt & 2) == 2
    b2 = (nt & 4) == 4
    ys = [jnp.where(b0, ys[2 * i + 1], ys[2 * i]) for i in range(4)]
    ys = [jnp.where(b1, ys[2 * i + 1], ys[2 * i]) for i in range(2)]
    y = jnp.where(b2, ys[1], ys[0])
    onehot = (nt == jax.lax.broadcasted_iota(jnp.int32, (1, T), 1)).astype(jnp.float32)
    bias = jnp.dot(onehot, b_ref[...], preferred_element_type=jnp.float32)
    o_ref[...] = jnp.maximum(y + bias, 0.0)


def kernel(agg_msg, node_type, W_att, b_att):
    x = agg_msg.reshape(N, IN)
    nt = node_type.astype(jnp.int32).reshape(N, 1)
    out = pl.pallas_call(
        _agg_kernel,
        grid=(N // B,),
        in_specs=[
            pl.BlockSpec((B, 1), lambda i: (i, 0)),
            pl.BlockSpec((B, IN), lambda i: (i, 0)),
            pl.BlockSpec((T, IN, OUT), lambda i: (0, 0, 0)),
            pl.BlockSpec((T, OUT), lambda i: (0, 0)),
        ],
        out_specs=pl.BlockSpec((B, OUT), lambda i: (i, 0)),
        out_shape=jax.ShapeDtypeStruct((N, OUT), jnp.float32),
    )(nt, x, W_att, b_att)
    return out
